# baseline (speedup 1.0000x reference)
import jax
import jax.numpy as jnp
from jax.experimental import pallas as pl
from jax.experimental.pallas import tpu as pltpu

_SEQ = pltpu.CompilerParams(dimension_semantics=("arbitrary",))


def _block_rows(n, target):
    best = None
    for b in range(8, min(n, target) + 1, 8):
        if n % b == 0:
            best = b
    return best if best is not None else n


def _bf(a):
    return a.astype(jnp.bfloat16)


def _dot(a, b):
    return jnp.dot(a, b, preferred_element_type=jnp.float32)


def _dot_t(a, b):
    return jax.lax.dot_general(a, b, (((0,), (0,)), ((), ())),
                               preferred_element_type=jnp.float32)


def _hyper_body(hh_ref, x_ref, w1_ref, b1_ref, w2_ref, b2_ref,
                he_ref, z_ref, h0_ref):
    @pl.when(pl.program_id(0) == 0)
    def _proj():
        h0_ref[...] = _bf(_dot(x_ref[...], w1_ref[...]) + b1_ref[...])

    he = jnp.maximum(_dot(_bf(hh_ref[...]), h0_ref[...]), 0.0)
    he_ref[...] = he
    z_ref[...] = _dot(he, w2_ref[...]) + b2_ref[...]


def _hg_body(hgt_ref, z_ref, out_ref, acc_ref):
    k = pl.program_id(0)
    nk = pl.num_programs(0)
    part = _dot_t(_bf(hgt_ref[...]), _bf(z_ref[...]))

    @pl.when(k == 0)
    def _init():
        acc_ref[...] = part

    @pl.when(k != 0)
    def _accum():
        acc_ref[...] += part

    @pl.when(k == nk - 1)
    def _finish():
        out_ref[...] = _bf(jnp.maximum(acc_ref[...], 0.0))


def _main_body(gg_ref, x_ref, hgn_ref, wg_ref, bg_ref, wm_ref, bm_ref,
               fused_ref, g0_ref):
    i = pl.program_id(0)
    bm_blk, d = fused_ref.shape

    @pl.when(i == 0)
    def _proj():
        g0_ref[...] = _bf(_dot(x_ref[...], wg_ref[...]) + bg_ref[...])

    gnn = jnp.maximum(_dot(_bf(gg_ref[...]), g0_ref[...]), 0.0)
    x_blk = x_ref[pl.ds(i * bm_blk, bm_blk), :]
    fused = _dot(x_blk, wm_ref[0:d, :])
    fused += _dot(gnn, wm_ref[d:2 * d, :])
    fused += _dot(hgn_ref[...], _bf(wm_ref[2 * d:3 * d, :]))
    fused_ref[...] = fused + bm_ref[...]


def kernel(x, norm_GG, norm_HH, norm_HG, Wg, bg, W1, b1, W2, b2, Wm, bm):
    n, d = x.shape
    h = norm_HH.shape[0]
    f32 = jnp.float32
    bf16 = jnp.bfloat16
    bg2 = bg.reshape(1, d)
    b12 = b1.reshape(1, d)
    b22 = b2.reshape(1, d)
    bm2 = bm.reshape(1, d)
    hgt = jnp.transpose(norm_HG)

    bm_h = _block_rows(h, 200)
    bm_hg = _block_rows(h, 400)
    bm_main = _block_rows(n, 200)

    hyper_emb, z = pl.pallas_call(
        _hyper_body,
        grid=(h // bm_h,),
        in_specs=[
            pl.BlockSpec((bm_h, n), lambda i: (i, 0)),
            pl.BlockSpec((n, d), lambda i: (0, 0)),
            pl.BlockSpec((d, d), lambda i: (0, 0)),
            pl.BlockSpec((1, d), lambda i: (0, 0)),
            pl.BlockSpec((d, d), lambda i: (0, 0)),
            pl.BlockSpec((1, d), lambda i: (0, 0)),
        ],
        out_specs=[
            pl.BlockSpec((bm_h, d), lambda i: (i, 0)),
            pl.BlockSpec((bm_h, d), lambda i: (i, 0)),
        ],
        out_shape=[
            jax.ShapeDtypeStruct((h, d), f32),
            jax.ShapeDtypeStruct((h, d), f32),
        ],
        scratch_shapes=[pltpu.VMEM((n, d), bf16)],
        compiler_params=_SEQ,
    )(norm_HH, x, W1, b12, W2, b22)

    hgn = pl.pallas_call(
        _hg_body,
        grid=(h // bm_hg,),
        in_specs=[
            pl.BlockSpec((bm_hg, n), lambda k: (k, 0)),
            pl.BlockSpec((bm_hg, d), lambda k: (k, 0)),
        ],
        out_specs=pl.BlockSpec((n, d), lambda k: (0, 0)),
        out_shape=jax.ShapeDtypeStruct((n, d), bf16),
        scratch_shapes=[pltpu.VMEM((n, d), f32)],
        compiler_params=_SEQ,
    )(hgt, z)

    fused_emb = pl.pallas_call(
        _main_body,
        grid=(n // bm_main,),
        in_specs=[
            pl.BlockSpec((bm_main, n), lambda i: (i, 0)),
            pl.BlockSpec((n, d), lambda i: (0, 0)),
            pl.BlockSpec((bm_main, d), lambda i: (i, 0)),
            pl.BlockSpec((d, d), lambda i: (0, 0)),
            pl.BlockSpec((1, d), lambda i: (0, 0)),
            pl.BlockSpec((3 * d, d), lambda i: (0, 0)),
            pl.BlockSpec((1, d), lambda i: (0, 0)),
        ],
        out_specs=pl.BlockSpec((bm_main, d), lambda i: (i, 0)),
        out_shape=jax.ShapeDtypeStruct((n, d), f32),
        scratch_shapes=[pltpu.VMEM((n, d), bf16)],
        compiler_params=_SEQ,
    )(norm_GG, x, hgn, Wg, bg2, Wm, bm2)

    return (fused_emb, hyper_emb)


# speedup vs baseline: 1.0326x; 1.0326x over previous
import jax
import jax.numpy as jnp
from jax.experimental import pallas as pl
from jax.experimental.pallas import tpu as pltpu

_SEQ = pltpu.CompilerParams(dimension_semantics=("arbitrary",))


def _block_rows(n, target):
    best = None
    for b in range(8, min(n, target) + 1, 8):
        if n % b == 0:
            best = b
    return best if best is not None else n


def _bf(a):
    return a.astype(jnp.bfloat16)


def _dot(a, b):
    return jnp.dot(a, b, preferred_element_type=jnp.float32)


def _dot_t(a, b):
    return jax.lax.dot_general(a, b, (((0,), (0,)), ((), ())),
                               preferred_element_type=jnp.float32)


def _hyper_body(hh_ref, x_ref, w1_ref, b1_ref, w2_ref, b2_ref,
                he_ref, z_ref, h0_ref):
    @pl.when(pl.program_id(0) == 0)
    def _proj():
        h0_ref[...] = _bf(_dot(x_ref[...], w1_ref[...]) + b1_ref[...])

    he = jnp.maximum(_dot(_bf(hh_ref[...]), h0_ref[...]), 0.0)
    he_ref[...] = he
    z_ref[...] = _dot(he, w2_ref[...]) + b2_ref[...]


def _hg_body(hgt_ref, z_ref, out_ref, acc_ref):
    k = pl.program_id(0)
    nk = pl.num_programs(0)
    part = _dot_t(_bf(hgt_ref[...]), _bf(z_ref[...]))

    @pl.when(k == 0)
    def _init():
        acc_ref[...] = part

    @pl.when(k != 0)
    def _accum():
        acc_ref[...] += part

    @pl.when(k == nk - 1)
    def _finish():
        out_ref[...] = _bf(jnp.maximum(acc_ref[...], 0.0))


def _main_body(gg_ref, x_ref, hgn_ref, wg_ref, bg_ref, wm_ref, bm_ref,
               fused_ref, g0_ref):
    i = pl.program_id(0)
    bm_blk, d = fused_ref.shape

    @pl.when(i == 0)
    def _proj():
        g0_ref[...] = _bf(_dot(x_ref[...], wg_ref[...]) + bg_ref[...])

    gnn = jnp.maximum(_dot(_bf(gg_ref[...]), g0_ref[...]), 0.0)
    x_blk = x_ref[pl.ds(i * bm_blk, bm_blk), :]
    fused = _dot(x_blk, wm_ref[0:d, :])
    fused += _dot(gnn, wm_ref[d:2 * d, :])
    fused += _dot(hgn_ref[...], _bf(wm_ref[2 * d:3 * d, :]))
    fused_ref[...] = fused + bm_ref[...]


def kernel(x, norm_GG, norm_HH, norm_HG, Wg, bg, W1, b1, W2, b2, Wm, bm):
    n, d = x.shape
    h = norm_HH.shape[0]
    f32 = jnp.float32
    bf16 = jnp.bfloat16
    bg2 = bg.reshape(1, d)
    b12 = b1.reshape(1, d)
    b22 = b2.reshape(1, d)
    bm2 = bm.reshape(1, d)
    hgt = jnp.transpose(norm_HG)

    bm_h = _block_rows(h, 400)
    bm_hg = _block_rows(h, 400)
    bm_main = _block_rows(n, 400)

    hyper_emb, z = pl.pallas_call(
        _hyper_body,
        grid=(h // bm_h,),
        in_specs=[
            pl.BlockSpec((bm_h, n), lambda i: (i, 0)),
            pl.BlockSpec((n, d), lambda i: (0, 0)),
            pl.BlockSpec((d, d), lambda i: (0, 0)),
            pl.BlockSpec((1, d), lambda i: (0, 0)),
            pl.BlockSpec((d, d), lambda i: (0, 0)),
            pl.BlockSpec((1, d), lambda i: (0, 0)),
        ],
        out_specs=[
            pl.BlockSpec((bm_h, d), lambda i: (i, 0)),
            pl.BlockSpec((bm_h, d), lambda i: (i, 0)),
        ],
        out_shape=[
            jax.ShapeDtypeStruct((h, d), f32),
            jax.ShapeDtypeStruct((h, d), f32),
        ],
        scratch_shapes=[pltpu.VMEM((n, d), bf16)],
        compiler_params=_SEQ,
    )(norm_HH, x, W1, b12, W2, b22)

    hgn = pl.pallas_call(
        _hg_body,
        grid=(h // bm_hg,),
        in_specs=[
            pl.BlockSpec((bm_hg, n), lambda k: (k, 0)),
            pl.BlockSpec((bm_hg, d), lambda k: (k, 0)),
        ],
        out_specs=pl.BlockSpec((n, d), lambda k: (0, 0)),
        out_shape=jax.ShapeDtypeStruct((n, d), bf16),
        scratch_shapes=[pltpu.VMEM((n, d), f32)],
        compiler_params=_SEQ,
    )(hgt, z)

    fused_emb = pl.pallas_call(
        _main_body,
        grid=(n // bm_main,),
        in_specs=[
            pl.BlockSpec((bm_main, n), lambda i: (i, 0)),
            pl.BlockSpec((n, d), lambda i: (0, 0)),
            pl.BlockSpec((bm_main, d), lambda i: (i, 0)),
            pl.BlockSpec((d, d), lambda i: (0, 0)),
            pl.BlockSpec((1, d), lambda i: (0, 0)),
            pl.BlockSpec((3 * d, d), lambda i: (0, 0)),
            pl.BlockSpec((1, d), lambda i: (0, 0)),
        ],
        out_specs=pl.BlockSpec((bm_main, d), lambda i: (i, 0)),
        out_shape=jax.ShapeDtypeStruct((n, d), f32),
        scratch_shapes=[pltpu.VMEM((n, d), bf16)],
        compiler_params=_SEQ,
    )(norm_GG, x, hgn, Wg, bg2, Wm, bm2)

    return (fused_emb, hyper_emb)
